# baseline (speedup 1.0000x reference)
import functools

import jax
import jax.numpy as jnp
from jax import lax
from jax.experimental import pallas as pl
from jax.experimental.pallas import tpu as pltpu
from jax.experimental.pallas import tpu_sc as plsc

BATCH = 4096
SEQ_LEN = 200
WORD_DIM = 64

_NUM_CORES = 2
_NUM_SUBCORES = 16
_NUM_WORKERS = _NUM_CORES * _NUM_SUBCORES
_ROWS_PER_WORKER = BATCH // _NUM_WORKERS
_IDS_PER_WORKER = _ROWS_PER_WORKER * SEQ_LEN

_CHUNK0 = 128
_CHUNK1 = SEQ_LEN - _CHUNK0

_NBUF = 2
_QUADS = _ROWS_PER_WORKER // _NBUF


def _sc_kernel(ids_hbm, table_hbm, pos_hbm, out_hbm,
               idxall, pos_v, out_bufs, g_sems, st_sems):
    cid = lax.axis_index("c")
    sid = lax.axis_index("s")
    wid = sid * _NUM_CORES + cid
    flat_base = wid * _IDS_PER_WORKER

    pltpu.sync_copy(pos_hbm.at[pl.ds(0, SEQ_LEN)], pos_v)
    pltpu.sync_copy(ids_hbm.at[pl.ds(flat_base, _IDS_PER_WORKER)], idxall)

    def fire_gather(r_loc, j):
        o = r_loc * SEQ_LEN
        pltpu.async_copy(table_hbm.at[idxall.at[pl.ds(o, _CHUNK0)]],
                         out_bufs[j].at[pl.ds(0, _CHUNK0)], g_sems[j])
        pltpu.async_copy(table_hbm.at[idxall.at[pl.ds(o + _CHUNK0, _CHUNK1)]],
                         out_bufs[j].at[pl.ds(_CHUNK0, _CHUNK1)], g_sems[j])

    def wait_gather(j):
        pltpu.make_async_copy(table_hbm.at[pl.ds(0, SEQ_LEN), pl.ds(0, 128)],
                              out_bufs[j], g_sems[j]).wait()

    def fire_store(r_loc, j):
        pltpu.async_copy(out_bufs[j].at[pl.ds(0, SEQ_LEN), pl.ds(0, WORD_DIM)],
                         out_hbm.at[pl.ds(flat_base + r_loc * SEQ_LEN, SEQ_LEN),
                                    pl.ds(0, WORD_DIM)],
                         st_sems[j])

    def wait_store(j):
        pltpu.make_async_copy(out_bufs[j].at[pl.ds(0, SEQ_LEN),
                                             pl.ds(0, WORD_DIM)],
                              out_hbm.at[pl.ds(0, SEQ_LEN),
                                         pl.ds(0, WORD_DIM)],
                              st_sems[j]).wait()

    def add_pos(j):
        def body8(r8, c):
            r = r8 * 8
            for rr in range(8):
                for k in range(WORD_DIM // 16):
                    plsc.addupdate(out_bufs[j].at[r + rr, pl.ds(k * 16, 16)],
                                   pos_v[r + rr, pl.ds(k * 16, 16)])
            return c
        lax.fori_loop(0, SEQ_LEN // 8, body8, 0)

    for j in range(_NBUF):
        fire_gather(j, j)

    def quad_body(i, carry):
        for j in range(_NBUF):
            r = i * _NBUF + j
            wait_gather(j)
            add_pos(j)
            fire_store(r, j)
            if j >= 1:
                @pl.when(i < _QUADS - 1)
                def _():
                    wait_store(j - 1)
                    fire_gather(i * _NBUF + _NBUF + (j - 1), j - 1)

        @pl.when(i < _QUADS - 1)
        def _():
            wait_store(_NBUF - 1)
            fire_gather(i * _NBUF + _NBUF + (_NBUF - 1), _NBUF - 1)
        return carry

    lax.fori_loop(0, _QUADS, quad_body, 0)

    for j in range(_NBUF):
        wait_store(j)


def _wrapped(ids_hbm, table_hbm, pos_hbm, out_hbm,
             idxall, pos_v, b0, b1,
             g0, g1, s0, s1):
    _sc_kernel(ids_hbm, table_hbm, pos_hbm, out_hbm, idxall, pos_v,
               [b0, b1], [g0, g1], [s0, s1])


@jax.jit
def _run(ids_flat, word_table, pos_table):
    mesh = plsc.VectorSubcoreMesh(core_axis_name="c", subcore_axis_name="s")
    f = functools.partial(
        pl.kernel,
        mesh=mesh,
        out_type=jax.ShapeDtypeStruct((BATCH * SEQ_LEN, 128), jnp.float32),
        scratch_types=(
            [pltpu.VMEM((_IDS_PER_WORKER,), jnp.int32),
             pltpu.VMEM((SEQ_LEN, WORD_DIM), jnp.float32)]
            + [pltpu.VMEM((SEQ_LEN, 128), jnp.float32)] * _NBUF
            + [pltpu.SemaphoreType.DMA] * (2 * _NBUF)
        ),
        compiler_params=pltpu.CompilerParams(use_tc_tiling_on_sc=False),
    )(_wrapped)
    return f(ids_flat, word_table, pos_table)


def kernel(input_ids, word_table, pos_table):
    ids_flat = input_ids.reshape(-1).astype(jnp.int32)
    table_pad = jnp.pad(word_table, ((0, 0), (0, 128 - WORD_DIM)))
    out = _run(ids_flat, table_pad, pos_table)
    return out.reshape(BATCH, SEQ_LEN, 128)[:, :, :WORD_DIM]


# speedup vs baseline: 1.4704x; 1.4704x over previous
import functools

import jax
import jax.numpy as jnp
from jax import lax
from jax.experimental import pallas as pl
from jax.experimental.pallas import tpu as pltpu
from jax.experimental.pallas import tpu_sc as plsc

BATCH = 4096
SEQ_LEN = 200
WORD_DIM = 64

_NUM_CORES = 2
_NUM_SUBCORES = 16
_NUM_WORKERS = _NUM_CORES * _NUM_SUBCORES
_ROWS_PER_WORKER = BATCH // _NUM_WORKERS
_IDS_PER_WORKER = _ROWS_PER_WORKER * SEQ_LEN

_CHUNK0 = 128
_CHUNK1 = SEQ_LEN - _CHUNK0

_NBUF = 4
_QUADS = _ROWS_PER_WORKER // _NBUF


def _sc_kernel(ids_hbm, table_hbm, pos_hbm, out_hbm,
               idxall, pos_v, out_bufs, g_sems, st_sems):
    cid = lax.axis_index("c")
    sid = lax.axis_index("s")
    wid = sid * _NUM_CORES + cid
    flat_base = wid * _IDS_PER_WORKER

    pltpu.sync_copy(pos_hbm.at[pl.ds(0, SEQ_LEN)], pos_v)
    pltpu.sync_copy(ids_hbm.at[pl.ds(flat_base, _IDS_PER_WORKER)], idxall)

    def fire_gather(r_loc, j):
        o = r_loc * SEQ_LEN
        pltpu.async_copy(table_hbm.at[idxall.at[pl.ds(o, _CHUNK0)]],
                         out_bufs[j].at[pl.ds(0, _CHUNK0)], g_sems[j])
        pltpu.async_copy(table_hbm.at[idxall.at[pl.ds(o + _CHUNK0, _CHUNK1)]],
                         out_bufs[j].at[pl.ds(_CHUNK0, _CHUNK1)], g_sems[j])

    def wait_gather(j):
        pltpu.make_async_copy(table_hbm.at[pl.ds(0, SEQ_LEN)],
                              out_bufs[j], g_sems[j]).wait()

    def fire_store(r_loc, j):
        pltpu.async_copy(out_bufs[j],
                         out_hbm.at[pl.ds(flat_base + r_loc * SEQ_LEN, SEQ_LEN),
                                    pl.ds(0, WORD_DIM)],
                         st_sems[j])

    def wait_store(j):
        pltpu.make_async_copy(out_bufs[j],
                              out_hbm.at[pl.ds(0, SEQ_LEN),
                                         pl.ds(0, WORD_DIM)],
                              st_sems[j]).wait()

    def add_pos(j):
        def body8(r8, c):
            r = r8 * 8
            for rr in range(8):
                for k in range(WORD_DIM // 16):
                    plsc.addupdate(out_bufs[j].at[r + rr, pl.ds(k * 16, 16)],
                                   pos_v[r + rr, pl.ds(k * 16, 16)])
            return c
        lax.fori_loop(0, SEQ_LEN // 8, body8, 0)

    for j in range(_NBUF):
        fire_gather(j, j)

    def quad_body(i, carry):
        for j in range(_NBUF):
            r = i * _NBUF + j
            wait_gather(j)
            add_pos(j)
            fire_store(r, j)
            if j >= 1:
                @pl.when(i < _QUADS - 1)
                def _():
                    wait_store(j - 1)
                    fire_gather(i * _NBUF + _NBUF + (j - 1), j - 1)

        @pl.when(i < _QUADS - 1)
        def _():
            wait_store(_NBUF - 1)
            fire_gather(i * _NBUF + _NBUF + (_NBUF - 1), _NBUF - 1)
        return carry

    lax.fori_loop(0, _QUADS, quad_body, 0)

    for j in range(_NBUF):
        wait_store(j)


def _wrapped(ids_hbm, table_hbm, pos_hbm, out_hbm,
             idxall, pos_v, b0, b1, b2, b3,
             g0, g1, g2, g3, s0, s1, s2, s3):
    _sc_kernel(ids_hbm, table_hbm, pos_hbm, out_hbm, idxall, pos_v,
               [b0, b1, b2, b3], [g0, g1, g2, g3], [s0, s1, s2, s3])


@jax.jit
def _run(ids_flat, word_table, pos_table):
    mesh = plsc.VectorSubcoreMesh(core_axis_name="c", subcore_axis_name="s")
    f = functools.partial(
        pl.kernel,
        mesh=mesh,
        out_type=jax.ShapeDtypeStruct((BATCH * SEQ_LEN, 128), jnp.float32),
        scratch_types=(
            [pltpu.VMEM((_IDS_PER_WORKER,), jnp.int32),
             pltpu.VMEM((SEQ_LEN, WORD_DIM), jnp.float32)]
            + [pltpu.VMEM((SEQ_LEN, WORD_DIM), jnp.float32)] * _NBUF
            + [pltpu.SemaphoreType.DMA] * (2 * _NBUF)
        ),
        compiler_params=pltpu.CompilerParams(use_tc_tiling_on_sc=False),
    )(_wrapped)
    return f(ids_flat, word_table, pos_table)


def kernel(input_ids, word_table, pos_table):
    ids_flat = input_ids.reshape(-1).astype(jnp.int32)
    out = _run(ids_flat, word_table, pos_table)
    return out.reshape(BATCH, SEQ_LEN, 128)[:, :, :WORD_DIM]
